# baseline (speedup 1.0000x reference)
import functools

import jax
import jax.numpy as jnp
from jax import lax
from jax.experimental import pallas as pl
from jax.experimental.pallas import tpu as pltpu
from jax.experimental.pallas import tpu_sc as plsc

BATCH = 16384
LIST_LEN = 50
POSITIONS = 50
TABLE_PAD = 64

NC, NS, L = 2, 16, 16
NW = NC * NS
COLS_W = BATCH // NW
CCHUNK = 256
NCHUNK = COLS_W // CCHUNK
NSLOT = 2
WPR = CCHUNK // L


def _build():
    mesh = plsc.VectorSubcoreMesh(core_axis_name="c", subcore_axis_name="s")

    @functools.partial(
        pl.kernel,
        mesh=mesh,
        out_type=jax.ShapeDtypeStruct((LIST_LEN, BATCH), jnp.float32),
        compiler_params=pltpu.CompilerParams(needs_layout_passes=False),
        scratch_types=[
            pltpu.VMEM((TABLE_PAD,), jnp.float32),
            pltpu.VMEM((NSLOT, LIST_LEN, CCHUNK), jnp.int32),
            pltpu.VMEM((NSLOT, LIST_LEN, CCHUNK), jnp.float32),
            pltpu.SemaphoreType.DMA,
            pltpu.SemaphoreType.DMA,
            pltpu.SemaphoreType.DMA,
            pltpu.SemaphoreType.DMA,
        ],
    )
    def gather_kernel(
        pos_hbm, model_hbm, out_hbm, table_v, idx_v, vals_v,
        in_sem0, in_sem1, out_sem0, out_sem1,
    ):
        wid = lax.axis_index("s") * NC + lax.axis_index("c")
        col0 = wid * COLS_W
        in_sems = (in_sem0, in_sem1)
        out_sems = (out_sem0, out_sem1)

        def in_copy(ci, s):
            return pltpu.make_async_copy(
                pos_hbm.at[:, pl.ds(col0 + ci * CCHUNK, CCHUNK)],
                idx_v.at[s],
                in_sems[s],
            )

        def out_copy(ci, s):
            return pltpu.make_async_copy(
                vals_v.at[s],
                out_hbm.at[:, pl.ds(col0 + ci * CCHUNK, CCHUNK)],
                out_sems[s],
            )

        in_copy(0, 0).start()
        in_copy(1, 1).start()
        pltpu.sync_copy(model_hbm, table_v.at[pl.ds(0, POSITIONS)])

        for ci in range(NCHUNK):
            s = ci % NSLOT
            in_copy(ci, s).wait()
            if ci >= NSLOT:
                out_copy(ci - NSLOT, s).wait()

            src = idx_v.at[s]
            dst = vals_v.at[s]

            @plsc.parallel_loop(0, LIST_LEN, unroll=4)
            def _row(r):
                for k in range(WPR):
                    idx = src[r, pl.ds(k * L, L)]
                    dst[r, pl.ds(k * L, L)] = plsc.load_gather(table_v, [idx])

            if ci + NSLOT < NCHUNK:
                in_copy(ci + NSLOT, s).start()
            out_copy(ci, s).start()

        for ci in range(max(NCHUNK - NSLOT, 0), NCHUNK):
            out_copy(ci, ci % NSLOT).wait()

    return gather_kernel


_GATHER = _build()


@jax.jit
def kernel(position, model):
    out_t = _GATHER(position.T, model)
    return out_t.T


# speedup vs baseline: 1.0351x; 1.0351x over previous
import functools

import jax
import jax.numpy as jnp
from jax import lax
from jax.experimental import pallas as pl
from jax.experimental.pallas import tpu as pltpu
from jax.experimental.pallas import tpu_sc as plsc

BATCH = 16384
LIST_LEN = 50
POSITIONS = 50
TABLE_PAD = 64

NC, NS, L = 2, 16, 16
NW = NC * NS
COLS_W = BATCH // NW
CCHUNK = 256
NCHUNK = COLS_W // CCHUNK
NSLOT = 2
WPR = CCHUNK // L


def _build():
    mesh = plsc.VectorSubcoreMesh(core_axis_name="c", subcore_axis_name="s")

    @functools.partial(
        pl.kernel,
        mesh=mesh,
        out_type=jax.ShapeDtypeStruct((LIST_LEN, BATCH), jnp.float32),
        compiler_params=pltpu.CompilerParams(needs_layout_passes=False),
        scratch_types=[
            pltpu.VMEM((TABLE_PAD,), jnp.float32),
            pltpu.VMEM((NSLOT, LIST_LEN, CCHUNK), jnp.int32),
            pltpu.VMEM((NSLOT, LIST_LEN, CCHUNK), jnp.float32),
            pltpu.SemaphoreType.DMA,
            pltpu.SemaphoreType.DMA,
            pltpu.SemaphoreType.DMA,
            pltpu.SemaphoreType.DMA,
        ],
    )
    def gather_kernel(
        pos_hbm, model_hbm, out_hbm, table_v, idx_v, vals_v,
        in_sem0, in_sem1, out_sem0, out_sem1,
    ):
        wid = lax.axis_index("s") * NC + lax.axis_index("c")
        col0 = wid * COLS_W
        in_sems = (in_sem0, in_sem1)
        out_sems = (out_sem0, out_sem1)

        def in_copy(ci, s):
            return pltpu.make_async_copy(
                pos_hbm.at[:, pl.ds(col0 + ci * CCHUNK, CCHUNK)],
                idx_v.at[s],
                in_sems[s],
            )

        def out_copy(ci, s):
            return pltpu.make_async_copy(
                vals_v.at[s],
                out_hbm.at[:, pl.ds(col0 + ci * CCHUNK, CCHUNK)],
                out_sems[s],
            )

        in_copy(0, 0).start()
        in_copy(1, 1).start()
        pltpu.sync_copy(model_hbm, table_v.at[pl.ds(0, POSITIONS)])

        for ci in range(NCHUNK):
            s = ci % NSLOT
            in_copy(ci, s).wait()
            if ci >= NSLOT:
                out_copy(ci - NSLOT, s).wait()

            src = idx_v.at[s]
            dst = vals_v.at[s]

            @plsc.parallel_loop(0, LIST_LEN, unroll=2)
            def _row(r):
                for k in range(WPR):
                    idx = src[r, pl.ds(k * L, L)]
                    dst[r, pl.ds(k * L, L)] = plsc.load_gather(table_v, [idx])

            if ci + NSLOT < NCHUNK:
                in_copy(ci + NSLOT, s).start()
            out_copy(ci, s).start()

        for ci in range(max(NCHUNK - NSLOT, 0), NCHUNK):
            out_copy(ci, ci % NSLOT).wait()

    return gather_kernel


_GATHER = _build()


@jax.jit
def kernel(position, model):
    out_t = _GATHER(position.T, model)
    return out_t.T
